# baseline (speedup 1.0000x reference)
import functools

import jax
import jax.numpy as jnp
from jax import lax
from jax.experimental import pallas as pl
from jax.experimental.pallas import tpu as pltpu
from jax.experimental.pallas import tpu_sc as plsc

VOCAB = 1000000
EMBED_DIM = 64
NC, NS = 2, 16
NW = NC * NS
GRP = 128
NBUF = 5
TBLK = 14336
NBLK = -(-VOCAB // TBLK)
REC_ROWS = NBLK * TBLK


def _make_records():
    def rec(tab_ref, out_ref):
        i0 = lax.broadcasted_iota(jnp.int32, (EMBED_DIM, 128), 0)
        i1 = lax.broadcasted_iota(jnp.int32, (EMBED_DIM, 128), 1)
        eye = (i0 == i1).astype(jnp.float32)
        out_ref[...] = lax.dot_general(
            tab_ref[...], eye, (((0,), (0,)), ((), ())),
            preferred_element_type=jnp.float32,
            precision=lax.Precision.HIGHEST,
        )

    return pl.pallas_call(
        rec,
        grid=(NBLK,),
        in_specs=[pl.BlockSpec((EMBED_DIM, TBLK), lambda i: (0, i))],
        out_specs=pl.BlockSpec((TBLK, 128), lambda i: (i, 0)),
        out_shape=jax.ShapeDtypeStruct((REC_ROWS, 128), jnp.float32),
    )


def _make_gather(n_rows):
    rows_per_w = n_rows // NW
    grps_per_w = rows_per_w // GRP
    n_iter = grps_per_w // NBUF
    assert n_rows % (NW * GRP) == 0 and grps_per_w % NBUF == 0

    mesh = plsc.VectorSubcoreMesh(core_axis_name="c", subcore_axis_name="s")

    @functools.partial(
        pl.kernel,
        out_type=jax.ShapeDtypeStruct((n_rows, 128), jnp.float32),
        mesh=mesh,
        scratch_types=[
            pltpu.VMEM((grps_per_w, GRP), jnp.int32),
            pltpu.VMEM((NBUF, GRP, 128), jnp.float32),
            pltpu.SemaphoreType.DMA,
            pltpu.SemaphoreType.DMA,
        ],
    )
    def emb(x_hbm, rec_hbm, out_hbm, idx_v, rows_v, gsem, osem):
        wid = lax.axis_index("s") * NC + lax.axis_index("c")
        grp0 = wid * grps_per_w

        pltpu.sync_copy(x_hbm.at[pl.ds(grp0, grps_per_w)], idx_v)

        def gcopy(g, j):
            return pltpu.make_async_copy(
                rec_hbm.at[idx_v.at[g]], rows_v.at[j], gsem
            )

        def ocopy(g, j):
            return pltpu.make_async_copy(
                rows_v.at[j], out_hbm.at[pl.ds((grp0 + g) * GRP, GRP)], osem
            )

        def body(i, carry):
            base = i * NBUF
            for j in range(NBUF):
                gcopy(base + j, j).start()
            for j in range(NBUF):
                gcopy(base + j, j).wait()
            for j in range(NBUF):
                ocopy(base + j, j).start()
            for j in range(NBUF):
                ocopy(base + j, j).wait()
            return carry

        lax.fori_loop(0, n_iter, body, 0)

    return emb


def kernel(x, table):
    b, s = x.shape
    n_rows = b * s
    x_flat = x.reshape(n_rows // GRP, GRP).astype(jnp.int32)
    records = _make_records()(table.T)
    out = _make_gather(n_rows)(x_flat, records)
    return out.reshape(b, s, 128)[:, :, :EMBED_DIM]
